# Initial kernel scaffold; baseline (speedup 1.0000x reference)
import jax
import jax.numpy as jnp
from jax.experimental import pallas as pl


def kernel(x, edge_index, batch, W1s, b1s, g1s, bt1s, W2s, b2s, bng, bnb, vn_emb, vW1, vb1, vg1, vbt1, vW2, vb2, vg2, vbt2):
    raise NotImplementedError("write your pallas kernel here")


# speedup vs baseline: 3.6583x; 3.6583x over previous
import functools

import jax
import jax.numpy as jnp
from jax import lax
from jax.experimental import pallas as pl
from jax.experimental.pallas import tpu as pltpu
from jax.experimental.pallas import tpu_sc as plsc

_CH = 128
_NZ = 64
_G = 128
_BN = 2000


def _sc_segsum_body(npad, n, d, ew, nchunk,
                    h_hbm, src_hbm, dst_hbm, out_hbm,
                    acc, zbuf, idx_s, idx_d, rows, sem):
  c = lax.axis_index("c")
  s = lax.axis_index("s")

  def zstore(i, _):
    r = i // (d // 16)
    col = (i % (d // 16)) * 16
    zbuf[r, pl.ds(col, 16)] = jnp.zeros((16,), jnp.float32)
    return 0
  lax.fori_loop(0, _NZ * (d // 16), zstore, 0)

  rows_per_sub = npad // 16
  def zcopy(i, _):
    pltpu.sync_copy(zbuf, acc.at[pl.ds(s * rows_per_sub + i * _NZ, _NZ)])
    return 0
  lax.fori_loop(0, rows_per_sub // _NZ, zcopy, 0)
  plsc.subcore_barrier()

  base = (c * 16 + s) * ew
  def chunk(k, _):
    off = base + k * _CH
    pltpu.sync_copy(src_hbm.at[pl.ds(off, _CH)], idx_s)
    pltpu.async_copy(h_hbm.at[idx_s], rows, sem).wait()
    pltpu.sync_copy(dst_hbm.at[pl.ds(off, _CH)], idx_d)
    pltpu.sync_copy(rows, acc.at[idx_d], add=True)
    return 0
  lax.fori_loop(0, nchunk, chunk, 0)
  plsc.subcore_barrier()

  outr = ((n + 15) // 16 + 7) // 8 * 8
  last = n - 15 * outr

  @pl.when(s < 15)
  def _():
    pltpu.sync_copy(acc.at[pl.ds(s * outr, outr)],
                    out_hbm.at[pl.ds(c * n + s * outr, outr)])

  @pl.when(s == 15)
  def _():
    pltpu.sync_copy(acc.at[pl.ds(15 * outr, last)],
                    out_hbm.at[pl.ds(c * n + 15 * outr, last)])


def _sc_segsum(h, srcp, dstp, npad):
  n, d = h.shape
  epad = srcp.shape[0]
  ew = epad // 32
  nchunk = ew // _CH
  mesh = plsc.VectorSubcoreMesh(core_axis_name="c", subcore_axis_name="s")
  f = pl.kernel(
      functools.partial(_sc_segsum_body, npad, n, d, ew, nchunk),
      out_type=jax.ShapeDtypeStruct((2 * n, d), jnp.float32),
      mesh=mesh,
      scratch_types=[
          pltpu.VMEM_SHARED((npad, d), jnp.float32),
          pltpu.VMEM((_NZ, d), jnp.float32),
          pltpu.VMEM((_CH,), jnp.int32),
          pltpu.VMEM((_CH,), jnp.int32),
          pltpu.VMEM((_CH, d), jnp.float32),
          pltpu.SemaphoreType.DMA,
      ],
  )
  return f(h, srcp, dstp)


def _mlp_body(relu_out, h_ref, ma_ref, mb_ref, w1_ref, b1_ref, g1_ref,
              t1_ref, w2_ref, b2_ref, bg_ref, bb_ref, row_ref, out_ref):
  z = h_ref[...] + ma_ref[...] + mb_ref[...]
  t = jnp.dot(z, w1_ref[...], preferred_element_type=jnp.float32)
  t = jnp.maximum((t + b1_ref[...]) * g1_ref[...] + t1_ref[...], 0.0)
  o = jnp.dot(t, w2_ref[...], preferred_element_type=jnp.float32)
  o = (o + b2_ref[...]) * bg_ref[...] + bb_ref[...]
  if relu_out:
    o = jnp.maximum(o, 0.0)
  out_ref[...] = o + row_ref[...]


def _tc_mlp(h, msg, w1, b1, g1, t1, w2, b2, bg, bb, row, relu_out):
  n, d = h.shape
  hd = w1.shape[1]
  grid = n // _BN
  nd = pl.BlockSpec((_BN, d), lambda i: (i, 0))
  mb = pl.BlockSpec((_BN, d), lambda i: (i + grid, 0))
  full = lambda shape: pl.BlockSpec(shape, lambda i: (0, 0))
  return pl.pallas_call(
      functools.partial(_mlp_body, relu_out),
      grid=(grid,),
      in_specs=[nd, nd, mb,
                full((d, hd)), full((1, hd)), full((1, hd)), full((1, hd)),
                full((hd, d)), full((1, d)), full((1, d)), full((1, d)),
                full((1, d))],
      out_specs=nd,
      out_shape=jax.ShapeDtypeStruct((n, d), jnp.float32),
  )(h, msg, msg, w1, b1, g1, t1, w2, b2, bg, bb, row)


def _pool_body(grid, h_ref, bf_ref, sum_ref, mean_ref, acc, cnt):
  i = pl.program_id(0)

  @pl.when(i == 0)
  def _():
    acc[...] = jnp.zeros_like(acc)
    cnt[...] = jnp.zeros_like(cnt)

  bn, d = h_ref.shape
  iota = lax.broadcasted_iota(jnp.int32, (bn, _G), 1).astype(jnp.float32)
  onehot = jnp.where(bf_ref[...] == iota, 1.0, 0.0)
  dims = (((0,), (0,)), ((), ()))
  acc[...] += lax.dot_general(onehot, h_ref[...], dims,
                              preferred_element_type=jnp.float32)
  cnt[...] += lax.dot_general(onehot, jnp.ones((bn, d), jnp.float32), dims,
                              preferred_element_type=jnp.float32)

  @pl.when(i == grid - 1)
  def _():
    sum_ref[...] = acc[...]
    mean_ref[...] = acc[...] / jnp.maximum(cnt[...], 1.0)


def _tc_pool(h, bf):
  n, d = h.shape
  grid = n // _BN
  out = jax.ShapeDtypeStruct((_G, d), jnp.float32)
  return pl.pallas_call(
      functools.partial(_pool_body, grid),
      grid=(grid,),
      in_specs=[pl.BlockSpec((_BN, d), lambda i: (i, 0)),
                pl.BlockSpec((_BN, 1), lambda i: (i, 0))],
      out_specs=[pl.BlockSpec((_G, d), lambda i: (0, 0))] * 2,
      out_shape=[out, out],
      scratch_shapes=[pltpu.VMEM((_G, d), jnp.float32),
                      pltpu.VMEM((_G, d), jnp.float32)],
  )(h, bf)


def _vmlp_body(p_ref, row_ref, w1_ref, b1_ref, g1_ref, t1_ref,
               w2_ref, b2_ref, g2_ref, t2_ref, out_ref):
  v = p_ref[...] + row_ref[...]
  t = jnp.dot(v, w1_ref[...], preferred_element_type=jnp.float32)
  t = jnp.maximum((t + b1_ref[...]) * g1_ref[...] + t1_ref[...], 0.0)
  o = jnp.dot(t, w2_ref[...], preferred_element_type=jnp.float32)
  o = jnp.maximum((o + b2_ref[...]) * g2_ref[...] + t2_ref[...], 0.0)
  out_ref[...] = o


def _tc_vmlp(p, row, w1, b1, g1, t1, w2, b2, g2, t2):
  return pl.pallas_call(
      _vmlp_body,
      out_shape=jax.ShapeDtypeStruct(p.shape, jnp.float32),
  )(p, row, w1, b1, g1, t1, w2, b2, g2, t2)


def _gadd_body(z_ref, bf_ref, v_ref, out_ref):
  bn, _ = z_ref.shape
  iota = lax.broadcasted_iota(jnp.int32, (bn, _G), 1).astype(jnp.float32)
  onehot = jnp.where(bf_ref[...] == iota, 1.0, 0.0)
  out_ref[...] = z_ref[...] + jnp.dot(onehot, v_ref[...],
                                      preferred_element_type=jnp.float32)


def _tc_gadd(z, bf, v):
  n, d = z.shape
  grid = n // _BN
  return pl.pallas_call(
      _gadd_body,
      grid=(grid,),
      in_specs=[pl.BlockSpec((_BN, d), lambda i: (i, 0)),
                pl.BlockSpec((_BN, 1), lambda i: (i, 0)),
                pl.BlockSpec((_G, d), lambda i: (0, 0))],
      out_specs=pl.BlockSpec((_BN, d), lambda i: (i, 0)),
      out_shape=jax.ShapeDtypeStruct((n, d), jnp.float32),
  )(z, bf, v)


def kernel(x, edge_index, batch, W1s, b1s, g1s, bt1s, W2s, b2s, bng, bnb,
           vn_emb, vW1, vb1, vg1, vbt1, vW2, vb2, vg2, vbt2):
  n, d = x.shape
  e = edge_index.shape[1]

  epw = 32 * _CH
  epad = ((e + epw - 1) // epw) * epw
  src = jnp.concatenate([edge_index[0], jnp.zeros((epad - e,), jnp.int32)])
  dst = jnp.concatenate([edge_index[1], jnp.full((epad - e,), n, jnp.int32)])
  unit = 16 * _NZ
  npad = ((n + 1 + unit - 1) // unit) * unit
  bf = batch.astype(jnp.float32).reshape(n, 1)
  r = lambda a: a.reshape(1, -1)
  zrow = jnp.zeros((1, d), jnp.float32)

  msg0 = _sc_segsum(x, src, dst, npad)
  h1 = _tc_mlp(x, msg0, W1s[0], r(b1s[0]), r(g1s[0]), r(bt1s[0]),
               W2s[0], r(b2s[0]), r(bng[0]), r(bnb[0]), vn_emb, True)
  msg1 = _sc_segsum(h1, src, dst, npad)
  z2 = _tc_mlp(h1, msg1, W1s[1], r(b1s[1]), r(g1s[1]), r(bt1s[1]),
               W2s[1], r(b2s[1]), r(bng[1]), r(bnb[1]), zrow, True)
  pooled2, _ = _tc_pool(z2, bf)
  v1 = _tc_vmlp(pooled2, vn_emb, vW1, r(vb1), r(vg1), r(vbt1),
                vW2, r(vb2), r(vg2), r(vbt2))
  h2 = _tc_gadd(z2, bf, v1)
  msg2 = _sc_segsum(h2, src, dst, npad)
  z3 = _tc_mlp(h2, msg2, W1s[2], r(b1s[2]), r(g1s[2]), r(bt1s[2]),
               W2s[2], r(b2s[2]), r(bng[2]), r(bnb[2]), zrow, False)
  _, readout = _tc_pool(z3, bf)
  return (z3, readout, v1)
